# baseline (speedup 1.0000x reference)
import functools

import jax
import jax.numpy as jnp
from jax import lax
from jax.experimental import pallas as pl
from jax.experimental.pallas import tpu as pltpu
from jax.experimental.pallas import tpu_sc as plsc

BATCH = 16384
EMBED = 64
NC, NS = 2, 16
NW = NC * NS
B_PER_W = BATCH // NW
CW = 128
NCHUNK = B_PER_W // CW


def _sc_gather(uidx, iidx, utab, itab):
    mesh = plsc.VectorSubcoreMesh(core_axis_name="c", subcore_axis_name="s")
    out = jax.ShapeDtypeStruct((BATCH, EMBED), jnp.float32)

    @functools.partial(
        pl.kernel,
        mesh=mesh,
        out_type=[out, out],
        scratch_types=[
            pltpu.VMEM((B_PER_W,), jnp.int32),
            pltpu.VMEM((B_PER_W,), jnp.int32),
            pltpu.SemaphoreType.DMA,
            pltpu.SemaphoreType.DMA,
        ],
    )
    def gather_k(uidx_hbm, iidx_hbm, utab_hbm, itab_hbm, uout_hbm, iout_hbm,
                 uidx_v, iidx_v, usem, isem):
        wid = lax.axis_index("s") * NC + lax.axis_index("c")
        base = wid * B_PER_W
        pltpu.sync_copy(uidx_hbm.at[pl.ds(base, B_PER_W)], uidx_v)
        pltpu.sync_copy(iidx_hbm.at[pl.ds(base, B_PER_W)], iidx_v)

        VL = 16

        def body(g, carry):
            off = g * VL
            uv = uidx_v[pl.ds(off, VL)]
            iv = iidx_v[pl.ds(off, VL)]
            for j in range(VL):
                pltpu.make_async_copy(
                    utab_hbm.at[pl.ds(uv[j], 1)],
                    uout_hbm.at[pl.ds(base + off + j, 1)],
                    usem).start()
                pltpu.make_async_copy(
                    itab_hbm.at[pl.ds(iv[j], 1)],
                    iout_hbm.at[pl.ds(base + off + j, 1)],
                    isem).start()
            return carry

        lax.fori_loop(0, B_PER_W // VL, body, 0)
        pltpu.make_async_copy(
            utab_hbm.at[pl.ds(0, B_PER_W)],
            uout_hbm.at[pl.ds(base, B_PER_W)], usem).wait()
        pltpu.make_async_copy(
            itab_hbm.at[pl.ds(0, B_PER_W)],
            iout_hbm.at[pl.ds(base, B_PER_W)], isem).wait()

    return gather_k(uidx, iidx, utab, itab)


def _mlp_body(u_ref, v_ref, w1u_ref, w1v_ref, b1_ref,
              w2_ref, b2_ref, w3_ref, b3_ref, w4_ref, b4_ref, o_ref):
    u = u_ref[...]
    v = v_ref[...]
    h = jnp.dot(u, w1u_ref[...], preferred_element_type=jnp.float32)
    h += jnp.dot(v, w1v_ref[...], preferred_element_type=jnp.float32)
    h = jnp.maximum(h + b1_ref[...], 0.0)
    h = jnp.dot(h, w2_ref[...], preferred_element_type=jnp.float32)
    h = jnp.maximum(h + b2_ref[...], 0.0)
    h = jnp.dot(h, w3_ref[...], preferred_element_type=jnp.float32)
    h = jnp.maximum(h + b3_ref[...], 0.0)
    z = jnp.dot(h, w4_ref[...], preferred_element_type=jnp.float32) + b4_ref[...]
    o_ref[...] = jax.nn.sigmoid(z)


def _tc_mlp(u_emb, i_emb, W1, b1, W2, b2, W3, b3, W4, b4):
    BB = 2048
    grid = (BATCH // BB,)
    w1u = W1[:, :EMBED].T
    w1v = W1[:, EMBED:].T
    w2t = W2.T
    w3t = W3.T
    w4t = W4.T
    full = lambda shape: pl.BlockSpec(shape, lambda i: (0, 0))
    out = pl.pallas_call(
        _mlp_body,
        grid=grid,
        in_specs=[
            pl.BlockSpec((BB, EMBED), lambda i: (i, 0)),
            pl.BlockSpec((BB, EMBED), lambda i: (i, 0)),
            full(w1u.shape),
            full(w1v.shape),
            full((1, 128)),
            full(w2t.shape),
            full((1, 64)),
            full(w3t.shape),
            full((1, 32)),
            full(w4t.shape),
            full((1, 1)),
        ],
        out_specs=pl.BlockSpec((BB, 1), lambda i: (i, 0)),
        out_shape=jax.ShapeDtypeStruct((BATCH, 1), jnp.float32),
    )(u_emb, i_emb, w1u, w1v, b1.reshape(1, -1), w2t,
      b2.reshape(1, -1), w3t, b3.reshape(1, -1), w4t, b4.reshape(1, 1))
    return jnp.squeeze(out, axis=-1)


def kernel(user_ids, item_ids, user_table, item_table,
           W1, b1, W2, b2, W3, b3, W4, b4):
    uids = user_ids.astype(jnp.int32)
    iids = item_ids.astype(jnp.int32)
    u_emb, i_emb = _sc_gather(uids, iids, user_table, item_table)
    return _tc_mlp(u_emb, i_emb, W1, b1, W2, b2, W3, b3, W4, b4)


# speedup vs baseline: 1.0660x; 1.0660x over previous
import functools

import jax
import jax.numpy as jnp
from jax import lax
from jax.experimental import pallas as pl
from jax.experimental.pallas import tpu as pltpu
from jax.experimental.pallas import tpu_sc as plsc

BATCH = 16384
EMBED = 64
NC, NS = 2, 16
NW = NC * NS
B_PER_W = BATCH // NW
CW = 128
NCHUNK = B_PER_W // CW


SLAB = 2 * EMBED
CW = 256
NCHUNK = B_PER_W // CW


def _sc_gather(uslab, islab, utab2, itab2):
    mesh = plsc.VectorSubcoreMesh(core_axis_name="c", subcore_axis_name="s")
    out = jax.ShapeDtypeStruct((BATCH, SLAB), jnp.float32)

    @functools.partial(
        pl.kernel,
        mesh=mesh,
        out_type=[out, out],
        scratch_types=[
            pltpu.VMEM((B_PER_W,), jnp.int32),
            pltpu.VMEM((B_PER_W,), jnp.int32),
            pltpu.VMEM((CW, SLAB), jnp.float32),
            pltpu.VMEM((CW, SLAB), jnp.float32),
            pltpu.SemaphoreType.DMA,
            pltpu.SemaphoreType.DMA,
        ],
    )
    def gather_k(uidx_hbm, iidx_hbm, utab_hbm, itab_hbm, uout_hbm, iout_hbm,
                 uidx_v, iidx_v, ubuf, ibuf, usem, isem):
        wid = lax.axis_index("s") * NC + lax.axis_index("c")
        base = wid * B_PER_W
        pltpu.sync_copy(uidx_hbm.at[pl.ds(base, B_PER_W)], uidx_v)
        pltpu.sync_copy(iidx_hbm.at[pl.ds(base, B_PER_W)], iidx_v)

        for c in range(NCHUNK):
            off = c * CW
            ucp = pltpu.make_async_copy(
                utab_hbm.at[uidx_v.at[pl.ds(off, CW)]], ubuf, usem)
            icp = pltpu.make_async_copy(
                itab_hbm.at[iidx_v.at[pl.ds(off, CW)]], ibuf, isem)
            ucp.start()
            icp.start()
            ucp.wait()
            pltpu.sync_copy(ubuf, uout_hbm.at[pl.ds(base + off, CW)])
            icp.wait()
            pltpu.sync_copy(ibuf, iout_hbm.at[pl.ds(base + off, CW)])

    return gather_k(uslab, islab, utab2, itab2)


def _mlp_body(u_ref, v_ref, up_ref, vp_ref, w1u_ref, w1v_ref, b1_ref,
              w2_ref, b2_ref, w3_ref, b3_ref, w4_ref, b4_ref, o_ref):
    up = up_ref[...] > 0
    vp = vp_ref[...] > 0
    u = jnp.where(up, u_ref[:, EMBED:], u_ref[:, :EMBED])
    v = jnp.where(vp, v_ref[:, EMBED:], v_ref[:, :EMBED])
    h = jnp.dot(u, w1u_ref[...], preferred_element_type=jnp.float32)
    h += jnp.dot(v, w1v_ref[...], preferred_element_type=jnp.float32)
    h = jnp.maximum(h + b1_ref[...], 0.0)
    h = jnp.dot(h, w2_ref[...], preferred_element_type=jnp.float32)
    h = jnp.maximum(h + b2_ref[...], 0.0)
    h = jnp.dot(h, w3_ref[...], preferred_element_type=jnp.float32)
    h = jnp.maximum(h + b3_ref[...], 0.0)
    z = jnp.dot(h, w4_ref[...], preferred_element_type=jnp.float32) + b4_ref[...]
    o_ref[...] = jax.nn.sigmoid(z)


def _tc_mlp(u_emb, i_emb, upar, ipar, W1, b1, W2, b2, W3, b3, W4, b4):
    BB = 2048
    grid = (BATCH // BB,)
    w1u = W1[:, :EMBED].T
    w1v = W1[:, EMBED:].T
    w2t = W2.T
    w3t = W3.T
    w4t = W4.T
    full = lambda shape: pl.BlockSpec(shape, lambda i: (0, 0))
    out = pl.pallas_call(
        _mlp_body,
        grid=grid,
        in_specs=[
            pl.BlockSpec((BB, SLAB), lambda i: (i, 0)),
            pl.BlockSpec((BB, SLAB), lambda i: (i, 0)),
            pl.BlockSpec((BB, 1), lambda i: (i, 0)),
            pl.BlockSpec((BB, 1), lambda i: (i, 0)),
            full(w1u.shape),
            full(w1v.shape),
            full((1, 128)),
            full(w2t.shape),
            full((1, 64)),
            full(w3t.shape),
            full((1, 32)),
            full(w4t.shape),
            full((1, 1)),
        ],
        out_specs=pl.BlockSpec((BB, 1), lambda i: (i, 0)),
        out_shape=jax.ShapeDtypeStruct((BATCH, 1), jnp.float32),
    )(u_emb, i_emb, upar.reshape(-1, 1), ipar.reshape(-1, 1),
      w1u, w1v, b1.reshape(1, -1), w2t,
      b2.reshape(1, -1), w3t, b3.reshape(1, -1), w4t, b4.reshape(1, 1))
    return jnp.squeeze(out, axis=-1)


def kernel(user_ids, item_ids, user_table, item_table,
           W1, b1, W2, b2, W3, b3, W4, b4):
    uids = user_ids.astype(jnp.int32)
    iids = item_ids.astype(jnp.int32)
    utab2 = user_table.reshape(-1, SLAB)
    itab2 = item_table.reshape(-1, SLAB)
    u_emb, i_emb = _sc_gather(uids >> 1, iids >> 1, utab2, itab2)
    return _tc_mlp(u_emb, i_emb, uids & 1, iids & 1,
                   W1, b1, W2, b2, W3, b3, W4, b4)
